# baseline (speedup 1.0000x reference)
import jax
import jax.numpy as jnp
from jax.experimental import pallas as pl

_K = 256
_SIGMA_N = 0.8
_EPS = 1e-8
_QT = 128
_SCW = 2048
_NBSA = 10
_NBS = 7


def _rimls_kernel(q_ref, pT_ref, nT_ref, f_ref, g_ref):
    qt = q_ref.shape[0]
    ns = pT_ref.shape[1]
    q = q_ref[...]
    pT = pT_ref[...]
    nT_raw = nT_ref[...]

    nnT = jnp.sqrt(jnp.sum(nT_raw * nT_raw, axis=0, keepdims=True))
    nT = nT_raw / jnp.maximum(nnT, _EPS)

    qx = q[:, 0:1]
    qy = q[:, 1:2]
    qz = q[:, 2:3]
    pxr, pyr, pzr = pT[0:1, :], pT[1:2, :], pT[2:3, :]
    nxr, nyr, nzr = nT[0:1, :], nT[1:2, :], nT[2:3, :]
    nn2_row = nxr * nxr + nyr * nyr + nzr * nzr
    c_row = pxr * nxr + pyr * nyr + pzr * nzr

    q2 = qx * qx + qy * qy + qz * qz
    s2 = pxr * pxr + pyr * pyr + pzr * pzr

    qp_b = jnp.dot(q.astype(jnp.bfloat16), pT.astype(jnp.bfloat16),
                   preferred_element_type=jnp.float32)
    d2 = jnp.maximum(q2 + s2 - 2.0 * qp_b, 0.0)

    kf = jnp.float32(_K)
    d2b = d2.astype(jnp.bfloat16)
    d2s = d2b[:, 0:2048]
    hiA = jnp.max(d2s, axis=1, keepdims=True)
    loA = jnp.zeros_like(hiA)

    def bsA(_, c):
        lo, hi = c
        mid = jnp.bfloat16(0.5) * (lo + hi)
        cnt = jnp.sum((d2s <= mid).astype(jnp.float32), axis=1, keepdims=True)
        ge = cnt >= kf
        return jnp.where(ge, lo, mid), jnp.where(ge, mid, hi)

    _, t_ub = jax.lax.fori_loop(0, _NBSA, bsA, (loA, hiA))

    def bsB(_, c):
        lo, hi = c
        mid = jnp.bfloat16(0.5) * (lo + hi)
        cnt = jnp.sum((d2b <= mid).astype(jnp.float32), axis=1, keepdims=True)
        ge = cnt >= kf
        return jnp.where(ge, lo, mid), jnp.where(ge, mid, hi)

    _, tb = jax.lax.fori_loop(0, _NBS, bsB, (jnp.zeros_like(t_ub), t_ub))
    T = tb.astype(jnp.float32)
    below = d2 < T
    cnt_lt = jnp.sum(below.astype(jnp.float32), axis=1, keepdims=True)
    sum_lt = jnp.sum(jnp.where(below, jnp.sqrt(d2), 0.0),
                     axis=1, keepdims=True)
    ksum = sum_lt + (kf - cnt_lt) * jnp.sqrt(T)
    h = ksum * (1.0 / _K) + _EPS

    rh2 = 1.0 / (h * h)
    isr = 1.0 / (0.5 * h + _EPS)
    isn2 = 1.0 / (_SIGMA_N * _SIGMA_N)
    nch = ns // _SCW

    fx_full = qx * nxr + qy * nyr + qz * nzr - c_row
    qp = qx * pxr + qy * pyr + qz * pzr
    r2_full = q2 + s2 - 2.0 * qp

    def fit_pass(prev):
        if prev is not None:
            f_p, gx_p, gy_p, gz_p = prev
            g2_p = gx_p * gx_p + gy_p * gy_p + gz_p * gz_p
        z = jnp.zeros((qt, 1), jnp.float32)
        acc = [z] * 11
        for c in range(nch):
            s = c * _SCW
            sl = slice(s, s + _SCW)
            n_x, n_y, n_z = nxr[:, sl], nyr[:, sl], nzr[:, sl]
            px = qx - pxr[:, sl]
            py = qy - pyr[:, sl]
            pz = qz - pzr[:, sl]
            fx = fx_full[:, sl]
            r2 = r2_full[:, sl]
            t = jnp.maximum(1.0 - r2 * rh2, 0.0)
            t2 = t * t
            phi = t2 * t2
            m = (-8.0 * rh2) * (t2 * t)
            if prev is None:
                w = phi
                cc = m
            else:
                u = (fx - f_p) * isr
                ng = n_x * gx_p + n_y * gy_p + n_z * gz_p
                a = jnp.exp(-(u * u)
                            - (nn2_row[:, sl] - 2.0 * ng + g2_p) * isn2)
                w = a * phi
                cc = a * m
            ex = cc * px
            ey = cc * py
            ez = cc * pz
            terms = (w, w * fx, ex, ey, ez, ex * fx, ey * fx, ez * fx,
                     w * n_x, w * n_y, w * n_z)
            acc = [a0 + jnp.sum(tm, axis=1, keepdims=True)
                   for a0, tm in zip(acc, terms)]
        sw, wfx, sex, sey, sez, sexf, seyf, sezf, wnx, wny, wnz = acc
        sumW = sw + _EPS
        f_new = wfx / sumW
        gx = (sexf - f_new * sex + wnx) / sumW
        gy = (seyf - f_new * sey + wny) / sumW
        gz = (sezf - f_new * sez + wnz) / sumW
        return f_new, gx, gy, gz

    out0 = fit_pass(None)
    f1, gx1, gy1, gz1 = fit_pass(out0)
    f_ref[...] = f1
    g_ref[:, 0:1] = gx1
    g_ref[:, 1:2] = gy1
    g_ref[:, 2:3] = gz1


def kernel(query_points, source_vertices, source_normals):
    nq = query_points.shape[0]
    ns = source_vertices.shape[0]
    pT = source_vertices.T
    nT = source_normals.T
    f2, g = pl.pallas_call(
        _rimls_kernel,
        grid=(nq // _QT,),
        in_specs=[
            pl.BlockSpec((_QT, 3), lambda i: (i, 0)),
            pl.BlockSpec((3, ns), lambda i: (0, 0)),
            pl.BlockSpec((3, ns), lambda i: (0, 0)),
        ],
        out_specs=[
            pl.BlockSpec((_QT, 1), lambda i: (i, 0)),
            pl.BlockSpec((_QT, 3), lambda i: (i, 0)),
        ],
        out_shape=[
            jax.ShapeDtypeStruct((nq, 1), jnp.float32),
            jax.ShapeDtypeStruct((nq, 3), jnp.float32),
        ],
    )(query_points, pT, nT)
    return f2[:, 0], g


# speedup vs baseline: 1.3374x; 1.3374x over previous
import jax
import jax.numpy as jnp
from jax.experimental import pallas as pl

_K = 256
_SIGMA_N = 0.8
_EPS = 1e-8
_QT = 128
_SCW = 2048
_NBSA = 10
_NBS = 7


def _rimls_kernel(q_ref, pT_ref, nT_ref, f_ref, g_ref):
    qt = q_ref.shape[0]
    ns = pT_ref.shape[1]
    q = q_ref[...]
    pT = pT_ref[...]
    nT_raw = nT_ref[...]

    nnT = jnp.sqrt(jnp.sum(nT_raw * nT_raw, axis=0, keepdims=True))
    nT = nT_raw / jnp.maximum(nnT, _EPS)

    qx = q[:, 0:1]
    qy = q[:, 1:2]
    qz = q[:, 2:3]
    pxr, pyr, pzr = pT[0:1, :], pT[1:2, :], pT[2:3, :]
    nxr, nyr, nzr = nT[0:1, :], nT[1:2, :], nT[2:3, :]
    nn2_row = nxr * nxr + nyr * nyr + nzr * nzr
    c_row = pxr * nxr + pyr * nyr + pzr * nzr

    q2 = qx * qx + qy * qy + qz * qz
    s2 = pxr * pxr + pyr * pyr + pzr * pzr

    qp_b = jnp.dot(q.astype(jnp.bfloat16), pT.astype(jnp.bfloat16),
                   preferred_element_type=jnp.float32)
    d2 = jnp.maximum(q2 + s2 - 2.0 * qp_b, 0.0)

    kf = jnp.float32(_K)
    d2s = d2[:, 0:2048]
    hiA = jnp.max(d2s, axis=1, keepdims=True)
    loA = jnp.zeros_like(hiA)

    def bsA(_, c):
        lo, hi = c
        mid = 0.5 * (lo + hi)
        cnt = jnp.sum((d2s <= mid).astype(jnp.float32), axis=1, keepdims=True)
        ge = cnt >= kf
        return jnp.where(ge, lo, mid), jnp.where(ge, mid, hi)

    _, t_ub = jax.lax.fori_loop(0, _NBSA, bsA, (loA, hiA))

    def bsB(_, c):
        lo, hi = c
        mid = 0.5 * (lo + hi)
        cnt = jnp.sum((d2 <= mid).astype(jnp.float32), axis=1, keepdims=True)
        ge = cnt >= kf
        return jnp.where(ge, lo, mid), jnp.where(ge, mid, hi)

    _, T = jax.lax.fori_loop(0, _NBS, bsB, (jnp.zeros_like(t_ub), t_ub))
    below = d2 < T
    cnt_lt = jnp.sum(below.astype(jnp.float32), axis=1, keepdims=True)
    sum_lt = jnp.sum(jnp.where(below, jnp.sqrt(d2), 0.0),
                     axis=1, keepdims=True)
    ksum = sum_lt + (kf - cnt_lt) * jnp.sqrt(T)
    h = ksum * (1.0 / _K) + _EPS

    rh2 = 1.0 / (h * h)
    isr = 1.0 / (0.5 * h + _EPS)
    isn2 = 1.0 / (_SIGMA_N * _SIGMA_N)
    nch = ns // _SCW

    fx_full = qx * nxr + qy * nyr + qz * nzr - c_row
    qp = qx * pxr + qy * pyr + qz * pzr
    r2_full = q2 + s2 - 2.0 * qp

    def fit_pass(prev):
        if prev is not None:
            f_p, gx_p, gy_p, gz_p = prev
            g2_p = gx_p * gx_p + gy_p * gy_p + gz_p * gz_p
        z = jnp.zeros((qt, 1), jnp.float32)
        acc = [z] * 11
        for c in range(nch):
            s = c * _SCW
            sl = slice(s, s + _SCW)
            n_x, n_y, n_z = nxr[:, sl], nyr[:, sl], nzr[:, sl]
            px = qx - pxr[:, sl]
            py = qy - pyr[:, sl]
            pz = qz - pzr[:, sl]
            fx = fx_full[:, sl]
            r2 = r2_full[:, sl]
            t = jnp.maximum(1.0 - r2 * rh2, 0.0)
            t2 = t * t
            phi = t2 * t2
            m = (-8.0 * rh2) * (t2 * t)
            if prev is None:
                w = phi
                cc = m
            else:
                u = (fx - f_p) * isr
                ng = n_x * gx_p + n_y * gy_p + n_z * gz_p
                a = jnp.exp(-(u * u)
                            - (nn2_row[:, sl] - 2.0 * ng + g2_p) * isn2)
                w = a * phi
                cc = a * m
            ex = cc * px
            ey = cc * py
            ez = cc * pz
            terms = (w, w * fx, ex, ey, ez, ex * fx, ey * fx, ez * fx,
                     w * n_x, w * n_y, w * n_z)
            acc = [a0 + jnp.sum(tm, axis=1, keepdims=True)
                   for a0, tm in zip(acc, terms)]
        sw, wfx, sex, sey, sez, sexf, seyf, sezf, wnx, wny, wnz = acc
        sumW = sw + _EPS
        f_new = wfx / sumW
        gx = (sexf - f_new * sex + wnx) / sumW
        gy = (seyf - f_new * sey + wny) / sumW
        gz = (sezf - f_new * sez + wnz) / sumW
        return f_new, gx, gy, gz

    out0 = fit_pass(None)
    f1, gx1, gy1, gz1 = fit_pass(out0)
    f_ref[...] = f1
    g_ref[:, 0:1] = gx1
    g_ref[:, 1:2] = gy1
    g_ref[:, 2:3] = gz1


def kernel(query_points, source_vertices, source_normals):
    nq = query_points.shape[0]
    ns = source_vertices.shape[0]
    pT = source_vertices.T
    nT = source_normals.T
    f2, g = pl.pallas_call(
        _rimls_kernel,
        grid=(nq // _QT,),
        in_specs=[
            pl.BlockSpec((_QT, 3), lambda i: (i, 0)),
            pl.BlockSpec((3, ns), lambda i: (0, 0)),
            pl.BlockSpec((3, ns), lambda i: (0, 0)),
        ],
        out_specs=[
            pl.BlockSpec((_QT, 1), lambda i: (i, 0)),
            pl.BlockSpec((_QT, 3), lambda i: (i, 0)),
        ],
        out_shape=[
            jax.ShapeDtypeStruct((nq, 1), jnp.float32),
            jax.ShapeDtypeStruct((nq, 3), jnp.float32),
        ],
    )(query_points, pT, nT)
    return f2[:, 0], g


# speedup vs baseline: 1.3441x; 1.0050x over previous
import jax
import jax.numpy as jnp
from jax.experimental import pallas as pl

_K = 256
_SIGMA_N = 0.8
_EPS = 1e-8
_QT = 128
_SCW = 2048
_NBSA = 9
_NBS = 7


def _rimls_kernel(q_ref, pT_ref, nT_ref, f_ref, g_ref):
    qt = q_ref.shape[0]
    ns = pT_ref.shape[1]
    q = q_ref[...]
    pT = pT_ref[...]
    nT_raw = nT_ref[...]

    nnT = jnp.sqrt(jnp.sum(nT_raw * nT_raw, axis=0, keepdims=True))
    nT = nT_raw / jnp.maximum(nnT, _EPS)

    qx = q[:, 0:1]
    qy = q[:, 1:2]
    qz = q[:, 2:3]
    pxr, pyr, pzr = pT[0:1, :], pT[1:2, :], pT[2:3, :]
    nxr, nyr, nzr = nT[0:1, :], nT[1:2, :], nT[2:3, :]
    nn2_row = nxr * nxr + nyr * nyr + nzr * nzr
    c_row = pxr * nxr + pyr * nyr + pzr * nzr

    q2 = qx * qx + qy * qy + qz * qz
    s2 = pxr * pxr + pyr * pyr + pzr * pzr

    qp_b = jnp.dot(q.astype(jnp.bfloat16), pT.astype(jnp.bfloat16),
                   preferred_element_type=jnp.float32)
    d2 = jnp.maximum(q2 + s2 - 2.0 * qp_b, 0.0)

    kf = jnp.float32(_K)
    d2s = d2[:, 0:2048]
    hiA = jnp.max(d2s, axis=1, keepdims=True)
    loA = jnp.zeros_like(hiA)

    def bsA(_, c):
        lo, hi = c
        mid = 0.5 * (lo + hi)
        cnt = jnp.sum((d2s <= mid).astype(jnp.float32), axis=1, keepdims=True)
        ge = cnt >= kf
        return jnp.where(ge, lo, mid), jnp.where(ge, mid, hi)

    _, t_ub = jax.lax.fori_loop(0, _NBSA, bsA, (loA, hiA))

    def bsB(_, c):
        lo, hi = c
        mid = 0.5 * (lo + hi)
        cnt = jnp.sum((d2 <= mid).astype(jnp.float32), axis=1, keepdims=True)
        ge = cnt >= kf
        return jnp.where(ge, lo, mid), jnp.where(ge, mid, hi)

    _, T = jax.lax.fori_loop(0, _NBS, bsB, (jnp.zeros_like(t_ub), t_ub))
    below = d2 < T
    cnt_lt = jnp.sum(below.astype(jnp.float32), axis=1, keepdims=True)
    sum_lt = jnp.sum(jnp.where(below, jnp.sqrt(d2), 0.0),
                     axis=1, keepdims=True)
    ksum = sum_lt + (kf - cnt_lt) * jnp.sqrt(T)
    h = ksum * (1.0 / _K) + _EPS

    rh2 = 1.0 / (h * h)
    isr = 1.0 / (0.5 * h + _EPS)
    isn2 = 1.0 / (_SIGMA_N * _SIGMA_N)
    nch = ns // _SCW

    fx_full = qx * nxr + qy * nyr + qz * nzr - c_row
    qp = qx * pxr + qy * pyr + qz * pzr
    t_full = jnp.maximum(1.0 - (q2 + s2 - 2.0 * qp) * rh2, 0.0)

    def fit_pass(prev):
        if prev is not None:
            f_p, gx_p, gy_p, gz_p = prev
            g2_p = gx_p * gx_p + gy_p * gy_p + gz_p * gz_p
        z = jnp.zeros((qt, 1), jnp.float32)
        acc = [z] * 11
        for c in range(nch):
            s = c * _SCW
            sl = slice(s, s + _SCW)
            n_x, n_y, n_z = nxr[:, sl], nyr[:, sl], nzr[:, sl]
            px = qx - pxr[:, sl]
            py = qy - pyr[:, sl]
            pz = qz - pzr[:, sl]
            fx = fx_full[:, sl]
            t = t_full[:, sl]
            t2 = t * t
            phi = t2 * t2
            m = (-8.0 * rh2) * (t2 * t)
            if prev is None:
                w = phi
                cc = m
            else:
                u = (fx - f_p) * isr
                ng = n_x * gx_p + n_y * gy_p + n_z * gz_p
                a = jnp.exp(-(u * u)
                            - (nn2_row[:, sl] - 2.0 * ng + g2_p) * isn2)
                w = a * phi
                cc = a * m
            ex = cc * px
            ey = cc * py
            ez = cc * pz
            terms = (w, w * fx, ex, ey, ez, ex * fx, ey * fx, ez * fx,
                     w * n_x, w * n_y, w * n_z)
            acc = [a0 + jnp.sum(tm, axis=1, keepdims=True)
                   for a0, tm in zip(acc, terms)]
        sw, wfx, sex, sey, sez, sexf, seyf, sezf, wnx, wny, wnz = acc
        sumW = sw + _EPS
        f_new = wfx / sumW
        gx = (sexf - f_new * sex + wnx) / sumW
        gy = (seyf - f_new * sey + wny) / sumW
        gz = (sezf - f_new * sez + wnz) / sumW
        return f_new, gx, gy, gz

    out0 = fit_pass(None)
    f1, gx1, gy1, gz1 = fit_pass(out0)
    f_ref[...] = f1
    g_ref[:, 0:1] = gx1
    g_ref[:, 1:2] = gy1
    g_ref[:, 2:3] = gz1


def kernel(query_points, source_vertices, source_normals):
    nq = query_points.shape[0]
    ns = source_vertices.shape[0]
    pT = source_vertices.T
    nT = source_normals.T
    f2, g = pl.pallas_call(
        _rimls_kernel,
        grid=(nq // _QT,),
        in_specs=[
            pl.BlockSpec((_QT, 3), lambda i: (i, 0)),
            pl.BlockSpec((3, ns), lambda i: (0, 0)),
            pl.BlockSpec((3, ns), lambda i: (0, 0)),
        ],
        out_specs=[
            pl.BlockSpec((_QT, 1), lambda i: (i, 0)),
            pl.BlockSpec((_QT, 3), lambda i: (i, 0)),
        ],
        out_shape=[
            jax.ShapeDtypeStruct((nq, 1), jnp.float32),
            jax.ShapeDtypeStruct((nq, 3), jnp.float32),
        ],
    )(query_points, pT, nT)
    return f2[:, 0], g


# speedup vs baseline: 1.3446x; 1.0004x over previous
import jax
import jax.numpy as jnp
from jax.experimental import pallas as pl

_K = 256
_SIGMA_N = 0.8
_EPS = 1e-8
_QT = 128
_SCW = 2048
_NBSA = 9
_NBS = 7


def _rimls_kernel(q_ref, pT_ref, nT_ref, f_ref, g_ref):
    qt = q_ref.shape[0]
    ns = pT_ref.shape[1]
    q = q_ref[...]
    pT = pT_ref[...]
    nT_raw = nT_ref[...]

    nnT = jnp.sqrt(jnp.sum(nT_raw * nT_raw, axis=0, keepdims=True))
    nT = nT_raw / jnp.maximum(nnT, _EPS)

    qx = q[:, 0:1]
    qy = q[:, 1:2]
    qz = q[:, 2:3]
    pxr, pyr, pzr = pT[0:1, :], pT[1:2, :], pT[2:3, :]
    nxr, nyr, nzr = nT[0:1, :], nT[1:2, :], nT[2:3, :]
    nn2_row = nxr * nxr + nyr * nyr + nzr * nzr
    c_row = pxr * nxr + pyr * nyr + pzr * nzr

    q2 = qx * qx + qy * qy + qz * qz
    s2 = pxr * pxr + pyr * pyr + pzr * pzr

    qp_b = jnp.dot(q.astype(jnp.bfloat16), pT.astype(jnp.bfloat16),
                   preferred_element_type=jnp.float32)
    d2 = jnp.maximum(q2 + s2 - 2.0 * qp_b, 0.0)

    kf = jnp.float32(_K)
    d2s = d2[:, 0:max(_K, min(2048, ns))]
    hiA = jnp.max(d2s, axis=1, keepdims=True)
    loA = jnp.zeros_like(hiA)

    def bsA(_, c):
        lo, hi = c
        mid = 0.5 * (lo + hi)
        cnt = jnp.sum((d2s <= mid).astype(jnp.float32), axis=1, keepdims=True)
        ge = cnt >= kf
        return jnp.where(ge, lo, mid), jnp.where(ge, mid, hi)

    _, t_ub = jax.lax.fori_loop(0, _NBSA, bsA, (loA, hiA))

    def bsB(_, c):
        lo, hi = c
        mid = 0.5 * (lo + hi)
        cnt = jnp.sum((d2 <= mid).astype(jnp.float32), axis=1, keepdims=True)
        ge = cnt >= kf
        return jnp.where(ge, lo, mid), jnp.where(ge, mid, hi)

    _, T = jax.lax.fori_loop(0, _NBS, bsB, (jnp.zeros_like(t_ub), t_ub))
    below = d2 < T
    cnt_lt = jnp.sum(below.astype(jnp.float32), axis=1, keepdims=True)
    sum_lt = jnp.sum(jnp.where(below, jnp.sqrt(d2), 0.0),
                     axis=1, keepdims=True)
    ksum = sum_lt + (kf - cnt_lt) * jnp.sqrt(T)
    h = ksum * (1.0 / _K) + _EPS

    rh2 = 1.0 / (h * h)
    isr = 1.0 / (0.5 * h + _EPS)
    isn2 = 1.0 / (_SIGMA_N * _SIGMA_N)
    nch = ns // _SCW

    fx_full = qx * nxr + qy * nyr + qz * nzr - c_row
    qp = qx * pxr + qy * pyr + qz * pzr
    t_full = jnp.maximum(1.0 - (q2 + s2 - 2.0 * qp) * rh2, 0.0)

    def fit_pass(prev):
        if prev is not None:
            f_p, gx_p, gy_p, gz_p = prev
            g2_p = gx_p * gx_p + gy_p * gy_p + gz_p * gz_p
        z = jnp.zeros((qt, 1), jnp.float32)
        acc = [z] * 11
        for c in range(nch):
            s = c * _SCW
            sl = slice(s, s + _SCW)
            n_x, n_y, n_z = nxr[:, sl], nyr[:, sl], nzr[:, sl]
            px = qx - pxr[:, sl]
            py = qy - pyr[:, sl]
            pz = qz - pzr[:, sl]
            fx = fx_full[:, sl]
            t = t_full[:, sl]
            t2 = t * t
            phi = t2 * t2
            m = (-8.0 * rh2) * (t2 * t)
            if prev is None:
                w = phi
                cc = m
            else:
                u = (fx - f_p) * isr
                ng = n_x * gx_p + n_y * gy_p + n_z * gz_p
                a = jnp.exp(-(u * u)
                            - (nn2_row[:, sl] - 2.0 * ng + g2_p) * isn2)
                w = a * phi
                cc = a * m
            ex = cc * px
            ey = cc * py
            ez = cc * pz
            terms = (w, w * fx, ex, ey, ez, ex * fx, ey * fx, ez * fx,
                     w * n_x, w * n_y, w * n_z)
            acc = [a0 + jnp.sum(tm, axis=1, keepdims=True)
                   for a0, tm in zip(acc, terms)]
        sw, wfx, sex, sey, sez, sexf, seyf, sezf, wnx, wny, wnz = acc
        sumW = sw + _EPS
        f_new = wfx / sumW
        gx = (sexf - f_new * sex + wnx) / sumW
        gy = (seyf - f_new * sey + wny) / sumW
        gz = (sezf - f_new * sez + wnz) / sumW
        return f_new, gx, gy, gz

    out0 = fit_pass(None)
    f1, gx1, gy1, gz1 = fit_pass(out0)
    f_ref[...] = f1
    g_ref[:, 0:1] = gx1
    g_ref[:, 1:2] = gy1
    g_ref[:, 2:3] = gz1


def kernel(query_points, source_vertices, source_normals):
    nq = query_points.shape[0]
    ns = source_vertices.shape[0]
    pT = source_vertices.T
    nT = source_normals.T
    f2, g = pl.pallas_call(
        _rimls_kernel,
        grid=(nq // _QT,),
        in_specs=[
            pl.BlockSpec((_QT, 3), lambda i: (i, 0)),
            pl.BlockSpec((3, ns), lambda i: (0, 0)),
            pl.BlockSpec((3, ns), lambda i: (0, 0)),
        ],
        out_specs=[
            pl.BlockSpec((_QT, 1), lambda i: (i, 0)),
            pl.BlockSpec((_QT, 3), lambda i: (i, 0)),
        ],
        out_shape=[
            jax.ShapeDtypeStruct((nq, 1), jnp.float32),
            jax.ShapeDtypeStruct((nq, 3), jnp.float32),
        ],
    )(query_points, pT, nT)
    return f2[:, 0], g
